# Initial kernel scaffold; baseline (speedup 1.0000x reference)
import jax
import jax.numpy as jnp
from jax.experimental import pallas as pl


def kernel(x, memory, Wq, bq, Wf, bf, gamma, beta):
    raise NotImplementedError("write your pallas kernel here")


# speedup vs baseline: 12.8365x; 12.8365x over previous
import jax
import jax.numpy as jnp
from jax import lax
from jax.experimental import pallas as pl
from jax.experimental.pallas import tpu as pltpu

TOPK = 8
EMBED_DIM = 1024
MEMORY_SIZE = 4096
TOKENS_PER_BLOCK = 256


def _fused_body(x_ref, mem_ref, wq_ref, bq_ref, wft_ref, wfb_ref, bf_ref,
                g_ref, b_ref, o_ref):
    xb = x_ref[...]
    q = jnp.dot(xb, wq_ref[...],
                preferred_element_type=jnp.float32) + bq_ref[...]
    s = lax.dot_general(q, mem_ref[...], (((1,), (1,)), ((), ())),
                        preferred_element_type=jnp.float32)
    scur = s
    t8 = None
    for _ in range(TOPK):
        t8 = jnp.max(scur, axis=1, keepdims=True)
        scur = jnp.where(scur == t8, -jnp.inf, scur)
    smax = jnp.max(s, axis=1, keepdims=True)
    w = jnp.where(s >= t8, jnp.exp(s - smax), 0.0)
    z = jnp.sum(w, axis=1, keepdims=True)
    mo = lax.dot_general(w, mem_ref[...], (((1,), (0,)), ((), ())),
                         preferred_element_type=jnp.float32) / z
    h = (jnp.dot(xb, wft_ref[...], preferred_element_type=jnp.float32)
         + jnp.dot(mo, wfb_ref[...], preferred_element_type=jnp.float32)
         + bf_ref[...])
    mean = jnp.mean(h, axis=1, keepdims=True)
    var = jnp.mean((h - mean) ** 2, axis=1, keepdims=True)
    hn = (h - mean) * lax.rsqrt(var + 1e-5) * g_ref[...] + b_ref[...]
    o_ref[...] = jnp.maximum(hn, 0.0)


def kernel(x, memory, Wq, bq, Wf, bf, gamma, beta):
    b, s, d = x.shape
    bs = b * s
    x2 = x.reshape(bs, d)
    wft = Wf[:d]
    wfb = Wf[d:]
    T = TOKENS_PER_BLOCK
    grid = (bs // T,)
    full = lambda i: (0, 0)
    out = pl.pallas_call(
        _fused_body,
        grid=grid,
        in_specs=[
            pl.BlockSpec((T, d), lambda i: (i, 0)),
            pl.BlockSpec((MEMORY_SIZE, d), full),
            pl.BlockSpec((d, d), full),
            pl.BlockSpec((1, d), full),
            pl.BlockSpec((d, d), full),
            pl.BlockSpec((d, d), full),
            pl.BlockSpec((1, d), full),
            pl.BlockSpec((1, d), full),
            pl.BlockSpec((1, d), full),
        ],
        out_specs=pl.BlockSpec((T, d), lambda i: (i, 0)),
        out_shape=jax.ShapeDtypeStruct((bs, d), jnp.float32),
        compiler_params=pltpu.CompilerParams(
            dimension_semantics=("arbitrary",),
        ),
    )(x2, memory, Wq, bq.reshape(1, d), wft, wfb, bf.reshape(1, d),
      gamma.reshape(1, d), beta.reshape(1, d))
    return out.reshape(b, s, d)
